# baseline (speedup 1.0000x reference)
import functools

import jax
import jax.numpy as jnp
from jax import lax
from jax.experimental import pallas as pl
from jax.experimental.pallas import tpu as pltpu
from jax.experimental.pallas import tpu_sc as plsc

VOCAB = 100000
EMB = 16
B = 1024
L = 20

NUM_CORES = 2
NUM_SUBCORES = 16
NW = NUM_CORES * NUM_SUBCORES
ROWS_W = B // NW
IDX_W = ROWS_W * L
IDX_CHUNK = 128
NCHUNK = IDX_W // IDX_CHUNK
NVEC = IDX_W // 16
NGRP = ROWS_W // 16
NQ = 4
QS = EMB // NQ

VB = 12800
NB = (VOCAB + VB - 1) // VB
VOCAB_PAD = NB * VB
EMB_AUG = EMB + 1
LOG2E = 1.4426950408889634


def _sc_gather_body(cw_hbm, tw_hbm, embt_hbm, wt_hbm, b_hbm,
                    cv_out, tl_out,
                    idx_v, tidx_v, idxe_v, tidxe_v, vals_v, wvals_v, bvals_v,
                    pool_v, tl_v,
                    sem_g, sem_w, sem_b):
    c = lax.axis_index("c")
    s = lax.axis_index("s")
    wid = s * NUM_CORES + c
    base = wid * ROWS_W

    pltpu.sync_copy(cw_hbm.at[pl.ds(wid * IDX_W, IDX_W)], idx_v)
    pltpu.sync_copy(tw_hbm.at[pl.ds(base, ROWS_W)], tidx_v)

    for q in range(NQ):
        off = jnp.int32(q * VOCAB)
        for cch in range(NVEC):
            sl = pl.ds(cch * 16, 16)
            idxe_v[q, sl] = idx_v[sl] + off
        for cch in range(NGRP):
            sl = pl.ds(cch * 16, 16)
            tidxe_v[q, sl] = tidx_v[sl] + off

    gathers = []
    for q in range(NQ):
        for j in range(NCHUNK):
            sl = pl.ds(j * IDX_CHUNK, IDX_CHUNK)
            gathers.append(
                pltpu.async_copy(embt_hbm.at[idxe_v.at[q, sl]],
                                 vals_v.at[q, sl], sem_g))
    wgathers = [pltpu.async_copy(wt_hbm.at[tidxe_v.at[q]], wvals_v.at[q],
                                 sem_w)
                for q in range(NQ)]
    gb = pltpu.async_copy(b_hbm.at[tidx_v], bvals_v, sem_b)
    for g in gathers:
        g.wait()
    for g in wgathers:
        g.wait()
    gb.wait()

    inv_l = jnp.float32(1.0 / L)
    lane = lax.iota(jnp.int32, 16)
    for g in range(NGRP):
        row16 = lane + jnp.int32(g * 16)
        tl_acc = bvals_v[pl.ds(g * 16, 16)]
        col0 = lane * jnp.int32(L) + jnp.int32(g * 16 * L)
        for q in range(NQ):
            qsplat = jnp.full((16,), q, jnp.int32)
            for s in range(QS):
                ssplat = jnp.full((16,), s, jnp.int32)
                acc = plsc.load_gather(vals_v, [qsplat, col0, ssplat])
                for l in range(1, L):
                    acc = acc + plsc.load_gather(
                        vals_v, [qsplat, col0 + jnp.int32(l), ssplat])
                cvv = acc * inv_l
                esplat = jnp.full((16,), q * QS + s, jnp.int32)
                plsc.store_scatter(pool_v, [row16, esplat], cvv)
                wv = plsc.load_gather(wvals_v, [qsplat, row16, ssplat])
                tl_acc = tl_acc + cvv * wv
        tl_v[pl.ds(g * 16, 16)] = tl_acc

    pltpu.sync_copy(pool_v, cv_out.at[pl.ds(base, ROWS_W)])
    pltpu.sync_copy(tl_v, tl_out.at[pl.ds(base, ROWS_W)])


_SC_GATHER_CACHE = []


def _sc_gather():
    if not _SC_GATHER_CACHE:
        _SC_GATHER_CACHE.append(functools.partial(
            pl.kernel,
            out_type=(
                jax.ShapeDtypeStruct((B, EMB), jnp.float32),
                jax.ShapeDtypeStruct((B,), jnp.float32),
            ),
            mesh=plsc.VectorSubcoreMesh(core_axis_name="c",
                                        subcore_axis_name="s",
                                        num_cores=NUM_CORES,
                                        num_subcores=NUM_SUBCORES),
            scratch_types=[
                pltpu.VMEM((IDX_W,), jnp.int32),
                pltpu.VMEM((ROWS_W,), jnp.int32),
                pltpu.VMEM((NQ, IDX_W), jnp.int32),
                pltpu.VMEM((NQ, ROWS_W), jnp.int32),
                pltpu.VMEM((NQ, IDX_W, QS), jnp.float32),
                pltpu.VMEM((NQ, ROWS_W, QS), jnp.float32),
                pltpu.VMEM((ROWS_W,), jnp.float32),
                pltpu.VMEM((ROWS_W, EMB), jnp.float32),
                pltpu.VMEM((ROWS_W,), jnp.float32),
                pltpu.SemaphoreType.DMA,
                pltpu.SemaphoreType.DMA,
                pltpu.SemaphoreType.DMA,
            ],
            compiler_params=pltpu.CompilerParams(use_tc_tiling_on_sc=False,
                                                 needs_layout_passes=False),
        )(_sc_gather_body))
    return _SC_GATHER_CACHE[0]


def _tc_loss_body(cv_ref, wtt_ref, tl_ref, out_ref, s_scr):
    i = pl.program_id(0)
    x = jnp.dot(cv_ref[...], wtt_ref[...],
                preferred_element_type=jnp.float32)
    ex = jnp.exp2(x)
    part = jnp.sum(ex, axis=1, keepdims=True)
    s_new = jnp.where(i == 0, part, s_scr[...] + part)
    s_scr[...] = s_new
    lse = jnp.log(s_new)
    out_ref[...] = (jnp.sum(lse - tl_ref[...], axis=(0, 1), keepdims=True)
                    * jnp.float32(1.0 / B))


_tc_loss = pl.pallas_call(
    _tc_loss_body,
    grid=(NB,),
    in_specs=[
        pl.BlockSpec((B, EMB_AUG), lambda i: (0, 0)),
        pl.BlockSpec((EMB_AUG, VB), lambda i: (0, i)),
        pl.BlockSpec((B, 1), lambda i: (0, 0)),
    ],
    out_specs=pl.BlockSpec((1, 1), lambda i: (0, 0)),
    out_shape=jax.ShapeDtypeStruct((1, 1), jnp.float32),
    scratch_shapes=[pltpu.VMEM((B, 1), jnp.float32)],
)


@jax.jit
def kernel(context_words, target_word, emb_table, W, b):
    cw = jnp.asarray(context_words, jnp.int32).reshape(B * L)
    tw = jnp.asarray(target_word, jnp.int32)
    embt = (emb_table.T.reshape(NQ, QS, VOCAB).transpose(0, 2, 1)
            .reshape(NQ * VOCAB, QS))
    wt = W.T.reshape(NQ, QS, VOCAB).transpose(0, 2, 1).reshape(NQ * VOCAB, QS)
    cv, tl = _sc_gather()(cw, tw, embt, wt, b)
    cv_aug = jnp.concatenate(
        [cv * jnp.float32(LOG2E), jnp.ones((B, 1), jnp.float32)],
        axis=1).astype(jnp.bfloat16)
    wpad = lax.pad(W.T, jnp.float32(0),
                   ((0, 0, 0), (0, VOCAB_PAD - VOCAB, 0)))
    bpad = lax.pad(b * jnp.float32(LOG2E), jnp.float32(-1e30),
                   [(0, VOCAB_PAD - VOCAB, 0)])
    wtt_aug = jnp.concatenate([wpad, bpad[None, :]],
                              axis=0).astype(jnp.bfloat16)
    loss = _tc_loss(cv_aug, wtt_aug, tl.reshape(B, 1))
    return loss[0, 0]


# speedup vs baseline: 10.6026x; 10.6026x over previous
import functools

import jax
import jax.numpy as jnp
from jax import lax
from jax.experimental import pallas as pl
from jax.experimental.pallas import tpu as pltpu
from jax.experimental.pallas import tpu_sc as plsc

VOCAB = 100000
EMB = 16
B = 1024
L = 20

NUM_CORES = 2
NUM_SUBCORES = 16
NW = NUM_CORES * NUM_SUBCORES
ROWS_W = B // NW
IDX_W = ROWS_W * L
IDX_CHUNK = 128
NCHUNK = IDX_W // IDX_CHUNK
NVEC = IDX_W // 16
NGRP = ROWS_W // 16

VB = 12800
NB = (VOCAB + VB - 1) // VB
VOCAB_PAD = NB * VB
EMB_AUG = EMB + 1
LOG2E = 1.4426950408889634
LN2 = 0.6931471805599453

def _sc_ctx_body(cw_hbm, embt_hbm, cv_out,
                 idx_v, idxe_v, vals_v, pool_v, sem_g):
    c = lax.axis_index("c")
    s = lax.axis_index("s")
    wid = s * NUM_CORES + c
    base = wid * ROWS_W

    pltpu.sync_copy(cw_hbm.at[pl.ds(wid * IDX_W, IDX_W)], idx_v)

    for e in range(EMB):
        off = jnp.int32(e * VOCAB)
        for cch in range(NVEC):
            sl = pl.ds(cch * 16, 16)
            idxe_v[e, sl] = idx_v[sl] + off

    gathers = []
    for e in range(EMB):
        for j in range(NCHUNK):
            sl = pl.ds(j * IDX_CHUNK, IDX_CHUNK)
            gathers.append(
                pltpu.async_copy(embt_hbm.at[idxe_v.at[e, sl]],
                                 vals_v.at[e, sl], sem_g))
    for g in gathers:
        g.wait()

    inv_l = jnp.float32(1.0 / L)
    lane = lax.iota(jnp.int32, 16)
    for g in range(NGRP):
        row16 = lane + jnp.int32(g * 16)
        col0 = lane * jnp.int32(L) + jnp.int32(g * 16 * L)
        for e in range(EMB):
            esplat = jnp.full((16,), e, jnp.int32)
            acc = plsc.load_gather(vals_v, [esplat, col0])
            for l in range(1, L):
                acc = acc + plsc.load_gather(
                    vals_v, [esplat, col0 + jnp.int32(l)])
            plsc.store_scatter(pool_v, [row16, esplat], acc * inv_l)

    pltpu.sync_copy(pool_v, cv_out.at[pl.ds(base, ROWS_W)])


def _sc_tgt_body(tw_hbm, wt_hbm, b_hbm, tw_out, bt_out,
                 tidx_v, tidxe_v, wvals_v, bvals_v, trow_v, sem_w, sem_b):
    c = lax.axis_index("c")
    s = lax.axis_index("s")
    wid = s * NUM_CORES + c
    base = wid * ROWS_W

    pltpu.sync_copy(tw_hbm.at[pl.ds(base, ROWS_W)], tidx_v)
    for e in range(EMB):
        off = jnp.int32(e * VOCAB)
        for cch in range(NGRP):
            sl = pl.ds(cch * 16, 16)
            tidxe_v[e, sl] = tidx_v[sl] + off

    wgathers = [pltpu.async_copy(wt_hbm.at[tidxe_v.at[e]], wvals_v.at[e],
                                 sem_w)
                for e in range(EMB)]
    gb = pltpu.async_copy(b_hbm.at[tidx_v], bvals_v, sem_b)
    for g in wgathers:
        g.wait()
    gb.wait()

    lane = lax.iota(jnp.int32, 16)
    for g in range(NGRP):
        row16 = lane + jnp.int32(g * 16)
        for e in range(EMB):
            esplat = jnp.full((16,), e, jnp.int32)
            plsc.store_scatter(trow_v, [row16, esplat],
                               wvals_v[e, pl.ds(g * 16, 16)])

    pltpu.sync_copy(trow_v, tw_out.at[pl.ds(base, ROWS_W)])
    pltpu.sync_copy(bvals_v, bt_out.at[pl.ds(base, ROWS_W)])


_SC_CACHE = {}


def _sc_kernels():
    if not _SC_CACHE:
        mesh = plsc.VectorSubcoreMesh(core_axis_name="c",
                                      subcore_axis_name="s",
                                      num_cores=NUM_CORES,
                                      num_subcores=NUM_SUBCORES)
        cparams = pltpu.CompilerParams(use_tc_tiling_on_sc=False,
                                       needs_layout_passes=False)
        _SC_CACHE["ctx"] = functools.partial(
            pl.kernel,
            out_type=jax.ShapeDtypeStruct((B, EMB), jnp.float32),
            mesh=mesh,
            scratch_types=[
                pltpu.VMEM((IDX_W,), jnp.int32),
                pltpu.VMEM((EMB, IDX_W), jnp.int32),
                pltpu.VMEM((EMB, IDX_W), jnp.float32),
                pltpu.VMEM((ROWS_W, EMB), jnp.float32),
                pltpu.SemaphoreType.DMA,
            ],
            compiler_params=cparams,
        )(_sc_ctx_body)
        _SC_CACHE["tgt"] = functools.partial(
            pl.kernel,
            out_type=(
                jax.ShapeDtypeStruct((B, EMB), jnp.float32),
                jax.ShapeDtypeStruct((B,), jnp.float32),
            ),
            mesh=mesh,
            scratch_types=[
                pltpu.VMEM((ROWS_W,), jnp.int32),
                pltpu.VMEM((EMB, ROWS_W), jnp.int32),
                pltpu.VMEM((EMB, ROWS_W), jnp.float32),
                pltpu.VMEM((ROWS_W,), jnp.float32),
                pltpu.VMEM((ROWS_W, EMB), jnp.float32),
                pltpu.SemaphoreType.DMA,
                pltpu.SemaphoreType.DMA,
            ],
            compiler_params=cparams,
        )(_sc_tgt_body)
    return _SC_CACHE["ctx"], _SC_CACHE["tgt"]


def _tc_loss_body(cv_ref, wtt_ref, twrows_ref, bt_ref, out_ref, s_scr):
    i = pl.program_id(0)
    x = jnp.dot(cv_ref[...], wtt_ref[...],
                preferred_element_type=jnp.float32)
    ex = jnp.exp2(x)
    part = jnp.sum(ex, axis=1, keepdims=True)
    s_new = jnp.where(i == 0, part, s_scr[...] + part)
    s_scr[...] = s_new
    lse = jnp.log(s_new)
    cvf = cv_ref[:, :EMB].astype(jnp.float32)
    tlogit = (jnp.sum(cvf * twrows_ref[...], axis=1, keepdims=True)
              * jnp.float32(LN2)
              + bt_ref[...])
    out_ref[...] = (jnp.sum(lse - tlogit, axis=(0, 1), keepdims=True)
                    * jnp.float32(1.0 / B))


_tc_loss = pl.pallas_call(
    _tc_loss_body,
    grid=(NB,),
    in_specs=[
        pl.BlockSpec((B, EMB_AUG), lambda i: (0, 0)),
        pl.BlockSpec((EMB_AUG, VB), lambda i: (0, i)),
        pl.BlockSpec((B, EMB), lambda i: (0, 0)),
        pl.BlockSpec((B, 1), lambda i: (0, 0)),
    ],
    out_specs=pl.BlockSpec((1, 1), lambda i: (0, 0)),
    out_shape=jax.ShapeDtypeStruct((1, 1), jnp.float32),
    scratch_shapes=[pltpu.VMEM((B, 1), jnp.float32)],
)


@jax.jit
def kernel(context_words, target_word, emb_table, W, b):
    cw = jnp.asarray(context_words, jnp.int32).reshape(B * L)
    tw = jnp.asarray(target_word, jnp.int32)
    embt = emb_table.T.reshape(EMB * VOCAB)
    wt = W.T.reshape(EMB * VOCAB)
    sc_ctx, sc_tgt = _sc_kernels()
    cv = sc_ctx(cw, embt)
    twrows, bt = sc_tgt(tw, wt, b)
    cv_aug = jnp.concatenate(
        [cv * jnp.float32(LOG2E), jnp.ones((B, 1), jnp.float32)],
        axis=1).astype(jnp.bfloat16)
    wpad = lax.pad(W.T, jnp.float32(0),
                   ((0, 0, 0), (0, VOCAB_PAD - VOCAB, 0)))
    bpad = lax.pad(b * jnp.float32(LOG2E), jnp.float32(-1e30),
                   [(0, VOCAB_PAD - VOCAB, 0)])
    wtt_aug = jnp.concatenate([wpad, bpad[None, :]],
                              axis=0).astype(jnp.bfloat16)
    loss = _tc_loss(cv_aug, wtt_aug, twrows, bt.reshape(B, 1))
    return loss[0, 0]


# speedup vs baseline: 10.6836x; 1.0076x over previous
import functools

import jax
import jax.numpy as jnp
from jax import lax
from jax.experimental import pallas as pl
from jax.experimental.pallas import tpu as pltpu
from jax.experimental.pallas import tpu_sc as plsc

VOCAB = 100000
EMB = 16
B = 1024
L = 20

NUM_CORES = 2
NUM_SUBCORES = 16
NW = NUM_CORES * NUM_SUBCORES
ROWS_W = B // NW
IDX_W = ROWS_W * L
IDX_CHUNK = 640
NCHUNK = IDX_W // IDX_CHUNK
NVEC = IDX_W // 16
NGRP = ROWS_W // 16

VB = 12800
NB = (VOCAB + VB - 1) // VB
VOCAB_PAD = NB * VB
EMB_AUG = EMB + 1
LOG2E = 1.4426950408889634
LN2 = 0.6931471805599453

def _sc_ctx_body(cw_hbm, embt_hbm, cv_out,
                 idx_v, idxe_v, vals_v, pool_v, sem_g):
    c = lax.axis_index("c")
    s = lax.axis_index("s")
    wid = s * NUM_CORES + c
    base = wid * ROWS_W

    pltpu.sync_copy(cw_hbm.at[pl.ds(wid * IDX_W, IDX_W)], idx_v)

    for e in range(EMB):
        off = jnp.int32(e * VOCAB)
        for cch in range(NVEC):
            sl = pl.ds(cch * 16, 16)
            idxe_v[e, sl] = idx_v[sl] + off

    gathers = []
    for e in range(EMB):
        for j in range(NCHUNK):
            sl = pl.ds(j * IDX_CHUNK, IDX_CHUNK)
            gathers.append(
                pltpu.async_copy(embt_hbm.at[idxe_v.at[e, sl]],
                                 vals_v.at[e, sl], sem_g))
    for g in gathers:
        g.wait()

    inv_l = jnp.float32(1.0 / L)
    lane = lax.iota(jnp.int32, 16)
    for g in range(NGRP):
        row16 = lane + jnp.int32(g * 16)
        col0 = lane * jnp.int32(L) + jnp.int32(g * 16 * L)
        for e in range(EMB):
            esplat = jnp.full((16,), e, jnp.int32)
            acc = plsc.load_gather(vals_v, [esplat, col0])
            for l in range(1, L):
                acc = acc + plsc.load_gather(
                    vals_v, [esplat, col0 + jnp.int32(l)])
            plsc.store_scatter(pool_v, [row16, esplat], acc * inv_l)

    pltpu.sync_copy(pool_v, cv_out.at[pl.ds(base, ROWS_W)])


def _sc_tgt_body(tw_hbm, wt_hbm, b_hbm, tw_out, bt_out,
                 tidx_v, tidxe_v, wvals_v, bvals_v, trow_v, sem_w, sem_b):
    c = lax.axis_index("c")
    s = lax.axis_index("s")
    wid = s * NUM_CORES + c
    base = wid * ROWS_W

    pltpu.sync_copy(tw_hbm.at[pl.ds(base, ROWS_W)], tidx_v)
    for e in range(EMB):
        off = jnp.int32(e * VOCAB)
        for cch in range(NGRP):
            sl = pl.ds(cch * 16, 16)
            tidxe_v[e, sl] = tidx_v[sl] + off

    wgathers = [pltpu.async_copy(wt_hbm.at[tidxe_v.at[e]], wvals_v.at[e],
                                 sem_w)
                for e in range(EMB)]
    gb = pltpu.async_copy(b_hbm.at[tidx_v], bvals_v, sem_b)
    for g in wgathers:
        g.wait()
    gb.wait()

    lane = lax.iota(jnp.int32, 16)
    for g in range(NGRP):
        row16 = lane + jnp.int32(g * 16)
        for e in range(EMB):
            esplat = jnp.full((16,), e, jnp.int32)
            plsc.store_scatter(trow_v, [row16, esplat],
                               wvals_v[e, pl.ds(g * 16, 16)])

    pltpu.sync_copy(trow_v, tw_out.at[pl.ds(base, ROWS_W)])
    pltpu.sync_copy(bvals_v, bt_out.at[pl.ds(base, ROWS_W)])


_SC_CACHE = {}


def _sc_kernels():
    if not _SC_CACHE:
        mesh = plsc.VectorSubcoreMesh(core_axis_name="c",
                                      subcore_axis_name="s",
                                      num_cores=NUM_CORES,
                                      num_subcores=NUM_SUBCORES)
        cparams = pltpu.CompilerParams(use_tc_tiling_on_sc=False,
                                       needs_layout_passes=False)
        _SC_CACHE["ctx"] = functools.partial(
            pl.kernel,
            out_type=jax.ShapeDtypeStruct((B, EMB), jnp.float32),
            mesh=mesh,
            scratch_types=[
                pltpu.VMEM((IDX_W,), jnp.int32),
                pltpu.VMEM((EMB, IDX_W), jnp.int32),
                pltpu.VMEM((EMB, IDX_W), jnp.float32),
                pltpu.VMEM((ROWS_W, EMB), jnp.float32),
                pltpu.SemaphoreType.DMA,
            ],
            compiler_params=cparams,
        )(_sc_ctx_body)
        _SC_CACHE["tgt"] = functools.partial(
            pl.kernel,
            out_type=(
                jax.ShapeDtypeStruct((B, EMB), jnp.float32),
                jax.ShapeDtypeStruct((B,), jnp.float32),
            ),
            mesh=mesh,
            scratch_types=[
                pltpu.VMEM((ROWS_W,), jnp.int32),
                pltpu.VMEM((EMB, ROWS_W), jnp.int32),
                pltpu.VMEM((EMB, ROWS_W), jnp.float32),
                pltpu.VMEM((ROWS_W,), jnp.float32),
                pltpu.VMEM((ROWS_W, EMB), jnp.float32),
                pltpu.SemaphoreType.DMA,
                pltpu.SemaphoreType.DMA,
            ],
            compiler_params=cparams,
        )(_sc_tgt_body)
    return _SC_CACHE["ctx"], _SC_CACHE["tgt"]


def _tc_loss_body(cv_ref, wtt_ref, twrows_ref, bt_ref, out_ref, s_scr):
    i = pl.program_id(0)
    x = jnp.dot(cv_ref[...], wtt_ref[...],
                preferred_element_type=jnp.float32)
    ex = jnp.exp2(x)
    part = jnp.sum(ex, axis=1, keepdims=True)
    s_new = jnp.where(i == 0, part, s_scr[...] + part)
    s_scr[...] = s_new
    lse = jnp.log(s_new)
    cvf = cv_ref[:, :EMB].astype(jnp.float32)
    tlogit = (jnp.sum(cvf * twrows_ref[...], axis=1, keepdims=True)
              * jnp.float32(LN2)
              + bt_ref[...])
    out_ref[...] = (jnp.sum(lse - tlogit, axis=(0, 1), keepdims=True)
                    * jnp.float32(1.0 / B))


_tc_loss = pl.pallas_call(
    _tc_loss_body,
    grid=(NB,),
    in_specs=[
        pl.BlockSpec((B, EMB_AUG), lambda i: (0, 0)),
        pl.BlockSpec((EMB_AUG, VB), lambda i: (0, i)),
        pl.BlockSpec((B, EMB), lambda i: (0, 0)),
        pl.BlockSpec((B, 1), lambda i: (0, 0)),
    ],
    out_specs=pl.BlockSpec((1, 1), lambda i: (0, 0)),
    out_shape=jax.ShapeDtypeStruct((1, 1), jnp.float32),
    scratch_shapes=[pltpu.VMEM((B, 1), jnp.float32)],
)


@jax.jit
def kernel(context_words, target_word, emb_table, W, b):
    cw = jnp.asarray(context_words, jnp.int32).reshape(B * L)
    tw = jnp.asarray(target_word, jnp.int32)
    embt = emb_table.T.reshape(EMB * VOCAB)
    wt = W.T.reshape(EMB * VOCAB)
    sc_ctx, sc_tgt = _sc_kernels()
    cv = sc_ctx(cw, embt)
    twrows, bt = sc_tgt(tw, wt, b)
    cv_aug = jnp.concatenate(
        [cv * jnp.float32(LOG2E), jnp.ones((B, 1), jnp.float32)],
        axis=1).astype(jnp.bfloat16)
    wpad = lax.pad(W.T, jnp.float32(0),
                   ((0, 0, 0), (0, VOCAB_PAD - VOCAB, 0)))
    bpad = lax.pad(b * jnp.float32(LOG2E), jnp.float32(-1e30),
                   [(0, VOCAB_PAD - VOCAB, 0)])
    wtt_aug = jnp.concatenate([wpad, bpad[None, :]],
                              axis=0).astype(jnp.bfloat16)
    loss = _tc_loss(cv_aug, wtt_aug, twrows, bt.reshape(B, 1))
    return loss[0, 0]
